# baseline (speedup 1.0000x reference)
import jax
import jax.numpy as jnp
from jax.experimental import pallas as pl
from jax.experimental.pallas import tpu as pltpu

R = 2048
C = 128
NTOT = R * C
LOGN = 18
NREAL = 384 * 384
RREAL = NREAL // C
PADROWS = R - RREAL
NSAMP = 16
SB = 8
RS = SB * R


def _roll_lanes(x, s):
    return jnp.concatenate([x[:, -s:], x[:, :-s]], axis=1)


def _shift_down(x, s):
    return jnp.concatenate([jnp.zeros((s, x.shape[1]), x.dtype), x[:-s]], axis=0)


def _lovasz_body(packed_ref, out_ref, key_ref):
    step = pl.program_id(0)

    packed = packed_ref[...].reshape(SB, RREAL, C)

    pad_key = jnp.int32(0x7F800000)
    blocks = []
    pad = jnp.full((PADROWS, C), pad_key, jnp.int32)
    for sidx in range(SB):
        blocks.append(packed[sidx])
        blocks.append(pad)
    key_ref[...] = jnp.concatenate(blocks, axis=0)

    row_iota = jax.lax.broadcasted_iota(jnp.int32, (RS, 1), 0)
    lane_iota = jax.lax.broadcasted_iota(jnp.int32, (1, C), 1)

    def exchange(pk, bitj, bitk):
        key = key_ref[...]
        keep_min = bitj == bitk
        take = (keep_min & (pk < key)) | (~keep_min & (pk > key))
        key_ref[...] = jnp.where(take, pk, key)

    def row_pass_small(j, k):
        s = 1 << j
        key = key_ref[...]
        bitj = (row_iota >> j) & 1
        is_upper = bitj == 1
        pk = jnp.where(is_upper, pltpu.roll(key, s, 0), pltpu.roll(key, RS - s, 0))
        bitk = _bitk_mask(k)
        exchange(pk, bitj, bitk)

    def row_pass_halves(j, k):
        s = 1 << j
        g = RS // (2 * s)
        v = key_ref[...].reshape(g, 2, s, C)
        a = v[:, 0]
        b = v[:, 1]
        if k < 11:
            asc = ((jax.lax.broadcasted_iota(jnp.int32, (g, 1, 1), 0)
                    >> (k - j - 1)) & 1) == 0
        elif k < LOGN:
            asc = ((jax.lax.broadcasted_iota(jnp.int32, (1, 1, C), 2)
                    >> (k - 11)) & 1) == 0
        else:
            asc = jnp.ones((1, 1, 1), jnp.bool_)
        mn = jnp.minimum(a, b)
        mx = jnp.maximum(a, b)
        na = jnp.where(asc, mn, mx)
        nb = jnp.where(asc, mx, mn)
        key_ref[...] = jnp.concatenate(
            [na[:, None], nb[:, None]], axis=1).reshape(RS, C)

    def _bitk_mask(k):
        if k < 11:
            return (row_iota >> k) & 1
        if k < LOGN:
            return (lane_iota >> (k - 11)) & 1
        return jnp.zeros((1, 1), jnp.int32)

    def lane_pass(j, k):
        s = 1 << (j - 11)
        key = key_ref[...]
        bitj = (lane_iota >> (j - 11)) & 1
        is_upper = bitj == 1
        pk = jnp.where(is_upper, _roll_lanes(key, s), _roll_lanes(key, -s))
        exchange(pk, bitj, _bitk_mask(k))

    for k in range(1, LOGN + 1):
        for j in range(k - 1, -1, -1):
            if j >= 11:
                lane_pass(j, k)
            elif j >= 3:
                row_pass_halves(j, k)
            else:
                row_pass_small(j, k)

    kp3 = key_ref[...].reshape(SB, R, C)
    lab = (kp3 & 1).astype(jnp.float32)
    bdec = jnp.where(kp3 >= 0, kp3, kp3 ^ jnp.int32(0x7FFFFFFF))
    e_sorted = -pltpu.bitcast(bdec, jnp.float32)

    csum = lab
    s = 1
    while s < R:
        csum = csum + jnp.concatenate(
            [jnp.zeros((SB, s, C), jnp.float32), csum[:, :-s, :]], axis=1)
        s *= 2
    tot = csum[:, R - 1:R, :]
    inc = tot
    s = 1
    while s < C:
        inc = inc + jnp.concatenate(
            [jnp.zeros((SB, 1, s), jnp.float32), inc[:, :, :-s]], axis=2)
        s *= 2
    kcum = csum + (inc - tot)
    p_total = inc[:, :, C - 1:C]

    pos = (jax.lax.broadcasted_iota(jnp.int32, (1, R, C), 1)
           + R * jax.lax.broadcasted_iota(jnp.int32, (1, R, C), 2)
           ).astype(jnp.float32)
    inter = p_total - kcum
    union = p_total + (pos + 1.0) - kcum
    jac = 1.0 - inter / (union + 1e-07)

    carry = jnp.concatenate(
        [jnp.zeros((SB, 1, 1), jnp.float32), jac[:, R - 1:R, :-1]], axis=2)
    jac_m1 = jnp.concatenate([carry, jac[:, :-1, :]], axis=1)

    loss_sum = jnp.sum(jnp.maximum(e_sorted, 0.0) * (jac - jac_m1))

    @pl.when(step == 0)
    def _():
        out_ref[...] = jnp.zeros((1, C), jnp.float32)

    out_ref[...] += jnp.full((1, C), loss_sum / NSAMP, jnp.float32)


@jax.jit
def kernel(pred, target):
    labi = target.astype(jnp.int32)
    signs = 2.0 * labi.astype(jnp.float32) - 1.0
    x = pred * signs - 1.0
    b = jax.lax.bitcast_convert_type(x, jnp.int32)
    mk = jnp.where(b >= 0, b, b ^ jnp.int32(0x7FFFFFFF))
    packed = (mk & jnp.int32(~1)) | labi

    out = pl.pallas_call(
        _lovasz_body,
        grid=(NSAMP // SB,),
        in_specs=[
            pl.BlockSpec((SB, 384, 384), lambda i: (i, 0, 0)),
        ],
        out_specs=pl.BlockSpec((1, C), lambda i: (0, 0)),
        out_shape=jax.ShapeDtypeStruct((1, C), jnp.float32),
        scratch_shapes=[
            pltpu.VMEM((RS, C), jnp.int32),
        ],
    )(packed)
    return out[0, 0]
